# baseline (speedup 1.0000x reference)
import functools

import jax
import jax.numpy as jnp
from jax import lax
from jax.experimental import pallas as pl
from jax.experimental.pallas import tpu as pltpu
from jax.experimental.pallas import tpu_sc as plsc

N_USERS = 10000
N_ENTITIES = 40000
N = N_USERS + N_ENTITIES
E = 800000
EMBED = 64
CF_LAMBDA = 1e-05
B = 1024

R_BLK = 2000
N_BLKS = N // R_BLK


def _leaky(x):
    return jnp.where(x >= 0, x, 0.01 * x)


def _l2n(x):
    n = jnp.sqrt(jnp.sum(x * x, axis=1, keepdims=True))
    return x / jnp.maximum(n, 1e-12)


def _dense0_body(ego_ref, p0_ref, p1_ref, p2_ref, p3_ref,
                 w1_ref, b1_ref, w2_ref, b2_ref,
                 e1_ref, re1_ref, im1_ref):
    ego = ego_ref[...]
    side = jnp.concatenate([p0_ref[0], p1_ref[0], p2_ref[0], p3_ref[0]], axis=1)
    h1 = _leaky(jnp.dot(ego + side, w1_ref[...],
                        preferred_element_type=jnp.float32) + b1_ref[...])
    h2 = _leaky(jnp.dot(ego * side, w2_ref[...],
                        preferred_element_type=jnp.float32) + b2_ref[...])
    e1 = h1 + h2
    eL = e1[:, :16]
    eR = e1[:, 16:]
    e1_ref[...] = jnp.stack([eL, eR], axis=0)
    re1_ref[...] = _l2n(eL)
    im1_ref[...] = _l2n(eR)


def _dense0(ego, side0, w1t, b1, w2t, b2):
    row = lambda i: (i, 0)
    full = lambda i: (0, 0)
    return pl.pallas_call(
        _dense0_body,
        grid=(N_BLKS,),
        in_specs=[
            pl.BlockSpec((R_BLK, 64), row),
            pl.BlockSpec((1, R_BLK, 16), lambda i: (0, i, 0)),
            pl.BlockSpec((1, R_BLK, 16), lambda i: (1, i, 0)),
            pl.BlockSpec((1, R_BLK, 16), lambda i: (2, i, 0)),
            pl.BlockSpec((1, R_BLK, 16), lambda i: (3, i, 0)),
            pl.BlockSpec((64, 32), full),
            pl.BlockSpec((1, 32), full),
            pl.BlockSpec((64, 32), full),
            pl.BlockSpec((1, 32), full),
        ],
        out_specs=[
            pl.BlockSpec((2, R_BLK, 16), lambda i: (0, i, 0)),
            pl.BlockSpec((R_BLK, 16), row),
            pl.BlockSpec((R_BLK, 16), row),
        ],
        out_shape=[
            jax.ShapeDtypeStruct((2, N, 16), jnp.float32),
            jax.ShapeDtypeStruct((N, 16), jnp.float32),
            jax.ShapeDtypeStruct((N, 16), jnp.float32),
        ],
    )(ego, side0, side0, side0, side0, w1t, b1, w2t, b2)


def _dense1_body(ego0_ref, ea_ref, eb_ref, sa_ref, sb_ref,
                 w1_ref, b1_ref, w2_ref, b2_ref, re1_ref, im1_ref, feat_ref):
    e1 = jnp.concatenate([ea_ref[0], eb_ref[0]], axis=1)
    s1 = jnp.concatenate([sa_ref[0], sb_ref[0]], axis=1)
    h1 = _leaky(jnp.dot(e1 + s1, w1_ref[...],
                        preferred_element_type=jnp.float32) + b1_ref[...])
    h2 = _leaky(jnp.dot(e1 * s1, w2_ref[...],
                        preferred_element_type=jnp.float32) + b2_ref[...])
    e2 = h1 + h2
    re2 = _l2n(e2[:, :8])
    im2 = _l2n(e2[:, 8:])
    ego0 = ego0_ref[...]
    z = jnp.zeros((ego0.shape[0], 8), jnp.float32)
    feat_ref[...] = jnp.concatenate(
        [ego0[:, :32], re1_ref[...], re2, z,
         ego0[:, 32:], im1_ref[...], im2, z], axis=1)


def _dense1(ego0, e1, side1, w1t, b1, w2t, b2, re1, im1):
    row = lambda i: (i, 0)
    full = lambda i: (0, 0)
    return pl.pallas_call(
        _dense1_body,
        grid=(N_BLKS,),
        in_specs=[
            pl.BlockSpec((R_BLK, 64), row),
            pl.BlockSpec((1, R_BLK, 16), lambda i: (0, i, 0)),
            pl.BlockSpec((1, R_BLK, 16), lambda i: (1, i, 0)),
            pl.BlockSpec((1, R_BLK, 16), lambda i: (0, i, 0)),
            pl.BlockSpec((1, R_BLK, 16), lambda i: (1, i, 0)),
            pl.BlockSpec((32, 16), full),
            pl.BlockSpec((1, 16), full),
            pl.BlockSpec((32, 16), full),
            pl.BlockSpec((1, 16), full),
            pl.BlockSpec((R_BLK, 16), row),
            pl.BlockSpec((R_BLK, 16), row),
        ],
        out_specs=pl.BlockSpec((R_BLK, 128), row),
        out_shape=jax.ShapeDtypeStruct((N, 128), jnp.float32),
    )(ego0, e1, e1, side1, side1, w1t, b1, w2t, b2, re1, im1)


def _loss_body(pos_ref, neg_ref, l2u_ref, l2p_ref, l2n_ref, out_ref):
    pos = jnp.sum(pos_ref[...], axis=1)
    neg = jnp.sum(neg_ref[...], axis=1)
    x = pos - neg
    nls = jnp.where(x >= 0, jnp.log1p(jnp.exp(-x)), -x + jnp.log1p(jnp.exp(x)))
    l2 = (jnp.mean(jnp.sum(l2u_ref[...], axis=1))
          + jnp.mean(jnp.sum(l2p_ref[...], axis=1))
          + jnp.mean(jnp.sum(l2n_ref[...], axis=1))) * 0.5
    out_ref[...] = (jnp.mean(nls) + CF_LAMBDA * l2).reshape(1, 1)


def _loss(pos_p, neg_p, l2u, l2p, l2n):
    return pl.pallas_call(
        _loss_body,
        out_shape=jax.ShapeDtypeStruct((1, 1), jnp.float32),
    )(pos_p, neg_p, l2u, l2p, l2n)


_CHUNK = 128
_SUP = 1024
_N_TILES = 16
_E_PAD = 819200
_PER_TILE = _E_PAD // _N_TILES
_N_SUP = _PER_TILE // _SUP
_IDX_ROWS = _E_PAD // _CHUNK
_VAL_ROWS = _E_PAD // 16
_ROWS_PER_TILE = 3128


def _copy_rows(src_ref, dst_ref, s):
    @pl.when(s < _N_TILES - 1)
    def _():
        r0 = pl.multiple_of(s * _ROWS_PER_TILE, 8)
        pltpu.sync_copy(src_ref.at[pl.ds(r0, _ROWS_PER_TILE)],
                        dst_ref.at[pl.ds(r0, _ROWS_PER_TILE)])

    @pl.when(s == _N_TILES - 1)
    def _():
        r0 = (_N_TILES - 1) * _ROWS_PER_TILE
        pltpu.sync_copy(src_ref.at[pl.ds(r0, N - r0)],
                        dst_ref.at[pl.ds(r0, N - r0)])


def _make_spmm_body(w):
    n_pass = w // 32

    def body(tabP, src2d, dst2d, val16, zeros, outP,
             srcA, dstA, valA, srcB, dstB, valB, rowsA, rowsB, accum,
             gsem, ssem, isem):
        c = lax.axis_index("c")
        s = lax.axis_index("s")
        bufA = (srcA, dstA, valA, rowsA)
        bufB = (srcB, dstB, valB, rowsB)
        base_i = s * (_PER_TILE // _CHUNK)
        base_v = s * (_PER_TILE // 16)

        def rslice(rows, j):
            return rows.at[pl.ds(pl.multiple_of(j * _CHUNK, _CHUNK), _CHUNK)]

        def one_pass(tab_q, out_q):
            _copy_rows(zeros, accum, s)
            plsc.subcore_barrier()

            def superchunk(sc, X, Y, first=False, last=False):
                (srcX, dstX, valX, rowsX) = X
                (srcY, dstY, valY, rowsY) = Y
                if not first:

                    def sdrain0(j, carry):
                        pltpu.make_async_copy(
                            rslice(rowsY, j), accum.at[dstY.at[j]],
                            ssem.at[j]).wait()
                        return carry

                    lax.fori_loop(0, _SUP // _CHUNK, sdrain0, 0)
                if not last:
                    ri = pl.multiple_of(base_i + (sc + 1) * (_SUP // _CHUNK), 8)
                    rv = pl.multiple_of(base_v + (sc + 1) * (_SUP // 16), 8)
                    pltpu.async_copy(src2d.at[pl.ds(ri, _SUP // _CHUNK)], srcY, isem)
                    pltpu.async_copy(dst2d.at[pl.ds(ri, _SUP // _CHUNK)], dstY, isem)
                    pltpu.async_copy(val16.at[pl.ds(rv, _SUP // 16)], valY, isem)

                def jbody(j, carry):
                    pltpu.make_async_copy(
                        tab_q.at[srcX.at[j]], rslice(rowsX, j),
                        gsem.at[j]).wait()

                    def grp(g2, carry2):
                        g = j * 8 + g2
                        vv = valX[g]
                        for i in range(16):
                            k = g * 16 + i
                            v = vv[i]
                            rowsX[k, pl.ds(0, 16)] = rowsX[k, pl.ds(0, 16)] * v
                        return carry2

                    lax.fori_loop(0, 8, grp, 0)
                    pltpu.make_async_copy(
                        rslice(rowsX, j), accum.at[dstX.at[j]],
                        ssem.at[j]).start(add=True)
                    if not last:
                        @pl.when(j == 1)
                        def _():
                            pltpu.make_async_copy(
                                src2d.at[pl.ds(ri, _SUP // _CHUNK)], srcY, isem).wait()
                            pltpu.make_async_copy(
                                dst2d.at[pl.ds(ri, _SUP // _CHUNK)], dstY, isem).wait()
                            pltpu.make_async_copy(
                                val16.at[pl.ds(rv, _SUP // 16)], valY, isem).wait()

                        @pl.when(j >= 1)
                        def _():
                            jp = j - 1
                            pltpu.async_copy(
                                tab_q.at[srcY.at[jp]], rslice(rowsY, jp),
                                gsem.at[jp])
                    return carry

                lax.fori_loop(0, _SUP // _CHUNK, jbody, 0)
                if not last:
                    _last = _SUP // _CHUNK - 1
                    pltpu.async_copy(
                        tab_q.at[srcY.at[_last]], rslice(rowsY, _last),
                        gsem.at[_last])

            r0 = pl.multiple_of(base_i, 8)
            pltpu.sync_copy(src2d.at[pl.ds(r0, _SUP // _CHUNK)], srcA)
            pltpu.sync_copy(dst2d.at[pl.ds(r0, _SUP // _CHUNK)], dstA)
            pltpu.sync_copy(
                val16.at[pl.ds(pl.multiple_of(base_v, 8), _SUP // 16)], valA)

            def gfire0(j, carry):
                pltpu.async_copy(
                    tab_q.at[srcA.at[j]], rslice(rowsA, j), gsem.at[j])
                return carry

            lax.fori_loop(0, _SUP // _CHUNK, gfire0, 0)

            superchunk(0, bufA, bufB, first=True)

            def dbl(dsc, carry):
                superchunk(2 * dsc + 1, bufB, bufA)
                superchunk(2 * dsc + 2, bufA, bufB)
                return carry

            lax.fori_loop(0, (_N_SUP - 2) // 2, dbl, 0)
            superchunk(_N_SUP - 1, bufB, bufA, last=True)

            def sdrain(j, carry):
                pltpu.make_async_copy(
                    rslice(rowsB, j), accum.at[dstB.at[j]], ssem.at[j]).wait()
                return carry

            lax.fori_loop(0, _SUP // _CHUNK, sdrain, 0)

            plsc.subcore_barrier()
            _copy_rows(accum, out_q, s)

        for p in range(n_pass):
            q = c * n_pass + p
            one_pass(tabP.at[q], outP.at[q])
            if p + 1 < n_pass:
                plsc.subcore_barrier()

    return body


@functools.partial(jax.jit, static_argnames=("w",))
def _spmm_sc(tabP, src2d, dst2d, val16, w):
    mesh = plsc.VectorSubcoreMesh(core_axis_name="c", subcore_axis_name="s")
    zeros = jnp.zeros((N, 16), jnp.float32)
    f = pl.kernel(
        _make_spmm_body(w),
        out_type=jax.ShapeDtypeStruct((w // 16, N, 16), jnp.float32),
        mesh=mesh,
        scratch_types=[
            pltpu.VMEM((_SUP // _CHUNK, _CHUNK), jnp.int32),
            pltpu.VMEM((_SUP // _CHUNK, _CHUNK), jnp.int32),
            pltpu.VMEM((_SUP // 16, 16), jnp.float32),
            pltpu.VMEM((_SUP // _CHUNK, _CHUNK), jnp.int32),
            pltpu.VMEM((_SUP // _CHUNK, _CHUNK), jnp.int32),
            pltpu.VMEM((_SUP // 16, 16), jnp.float32),
            pltpu.VMEM((_SUP, 16), jnp.float32),
            pltpu.VMEM((_SUP, 16), jnp.float32),
            pltpu.VMEM_SHARED((N, 16), jnp.float32),
            pltpu.SemaphoreType.DMA((_SUP // _CHUNK,)),
            pltpu.SemaphoreType.DMA((_SUP // _CHUNK,)),
            pltpu.SemaphoreType.DMA,
        ],
        compiler_params=pltpu.CompilerParams(use_tc_tiling_on_sc=False),
    )
    return f(tabP, src2d, dst2d, val16, zeros)


def _pad_edges(src, dst, val):
    pad = _E_PAD - E
    z = jnp.zeros((pad,), jnp.int32)
    src2d = jnp.concatenate([src, z]).reshape(_IDX_ROWS, _CHUNK)
    dst2d = jnp.concatenate([dst, z]).reshape(_IDX_ROWS, _CHUNK)
    val16 = jnp.concatenate(
        [val, jnp.zeros((pad,), jnp.float32)]).reshape(_VAL_ROWS, 16)
    return src2d, dst2d, val16


def _gather_partials(feat, uid, pid, nid):
    u = feat[uid]
    p = feat[pid]
    n = feat[nid]
    pos_p = (u * p).reshape(B, 8, 16).sum(axis=1)
    neg_p = (u * n).reshape(B, 8, 16).sum(axis=1)
    l2u = (u[:, :64] ** 2).reshape(B, 4, 16).sum(axis=1)
    l2p = (p[:, :64] ** 2).reshape(B, 4, 16).sum(axis=1)
    l2n = (n[:, :64] ** 2).reshape(B, 4, 16).sum(axis=1)
    return pos_p, neg_p, l2u, l2p, l2n


def kernel(user_ids, item_pos_ids, item_neg_ids, entity_user_embed,
           edge_src, edge_dst, edge_val,
           W1_0, b1_0, W2_0, b2_0, W1_1, b1_1, W2_1, b2_1):
    ego0 = entity_user_embed
    src2d, dst2d, val16 = _pad_edges(edge_src, edge_dst, edge_val)
    tab0 = ego0.reshape(N, 4, 16).transpose(1, 0, 2)
    side0 = _spmm_sc(tab0, src2d, dst2d, val16, 64)
    e1, re1, im1 = _dense0(
        ego0, side0, W1_0.T, b1_0.reshape(1, 32), W2_0.T, b2_0.reshape(1, 32))
    side1 = _spmm_sc(e1, src2d, dst2d, val16, 32)
    feat = _dense1(ego0, e1, side1,
                   W1_1.T, b1_1.reshape(1, 16), W2_1.T, b2_1.reshape(1, 16),
                   re1, im1)
    pos_p, neg_p, l2u, l2p, l2n = _gather_partials(
        feat, user_ids, item_pos_ids, item_neg_ids)
    out = _loss(pos_p, neg_p, l2u, l2p, l2n)
    return out.reshape(())


# speedup vs baseline: 1.0353x; 1.0353x over previous
import functools

import jax
import jax.numpy as jnp
from jax import lax
from jax.experimental import pallas as pl
from jax.experimental.pallas import tpu as pltpu
from jax.experimental.pallas import tpu_sc as plsc

N_USERS = 10000
N_ENTITIES = 40000
N = N_USERS + N_ENTITIES
E = 800000
EMBED = 64
CF_LAMBDA = 1e-05
B = 1024

R_BLK = 2000
N_BLKS = N // R_BLK


def _leaky(x):
    return jnp.where(x >= 0, x, 0.01 * x)


def _l2n(x):
    n = jnp.sqrt(jnp.sum(x * x, axis=1, keepdims=True))
    return x / jnp.maximum(n, 1e-12)


def _dense0_body(ego_ref, p0_ref, p1_ref, p2_ref, p3_ref,
                 w1_ref, b1_ref, w2_ref, b2_ref,
                 e1_ref, re1_ref, im1_ref):
    ego = ego_ref[...]
    side = jnp.concatenate([p0_ref[0], p1_ref[0], p2_ref[0], p3_ref[0]], axis=1)
    h1 = _leaky(jnp.dot(ego + side, w1_ref[...],
                        preferred_element_type=jnp.float32) + b1_ref[...])
    h2 = _leaky(jnp.dot(ego * side, w2_ref[...],
                        preferred_element_type=jnp.float32) + b2_ref[...])
    e1 = h1 + h2
    eL = e1[:, :16]
    eR = e1[:, 16:]
    e1_ref[...] = jnp.stack([eL, eR], axis=0)
    re1_ref[...] = _l2n(eL)
    im1_ref[...] = _l2n(eR)


def _dense0(ego, side0, w1t, b1, w2t, b2):
    row = lambda i: (i, 0)
    full = lambda i: (0, 0)
    return pl.pallas_call(
        _dense0_body,
        grid=(N_BLKS,),
        in_specs=[
            pl.BlockSpec((R_BLK, 64), row),
            pl.BlockSpec((1, R_BLK, 16), lambda i: (0, i, 0)),
            pl.BlockSpec((1, R_BLK, 16), lambda i: (1, i, 0)),
            pl.BlockSpec((1, R_BLK, 16), lambda i: (2, i, 0)),
            pl.BlockSpec((1, R_BLK, 16), lambda i: (3, i, 0)),
            pl.BlockSpec((64, 32), full),
            pl.BlockSpec((1, 32), full),
            pl.BlockSpec((64, 32), full),
            pl.BlockSpec((1, 32), full),
        ],
        out_specs=[
            pl.BlockSpec((2, R_BLK, 16), lambda i: (0, i, 0)),
            pl.BlockSpec((R_BLK, 16), row),
            pl.BlockSpec((R_BLK, 16), row),
        ],
        out_shape=[
            jax.ShapeDtypeStruct((2, N, 16), jnp.float32),
            jax.ShapeDtypeStruct((N, 16), jnp.float32),
            jax.ShapeDtypeStruct((N, 16), jnp.float32),
        ],
    )(ego, side0, side0, side0, side0, w1t, b1, w2t, b2)


def _dense1_body(ego0_ref, ea_ref, eb_ref, sa_ref, sb_ref,
                 w1_ref, b1_ref, w2_ref, b2_ref, re1_ref, im1_ref, feat_ref):
    e1 = jnp.concatenate([ea_ref[0], eb_ref[0]], axis=1)
    s1 = jnp.concatenate([sa_ref[0], sb_ref[0]], axis=1)
    h1 = _leaky(jnp.dot(e1 + s1, w1_ref[...],
                        preferred_element_type=jnp.float32) + b1_ref[...])
    h2 = _leaky(jnp.dot(e1 * s1, w2_ref[...],
                        preferred_element_type=jnp.float32) + b2_ref[...])
    e2 = h1 + h2
    re2 = _l2n(e2[:, :8])
    im2 = _l2n(e2[:, 8:])
    ego0 = ego0_ref[...]
    z = jnp.zeros((ego0.shape[0], 8), jnp.float32)
    feat_ref[...] = jnp.concatenate(
        [ego0[:, :32], re1_ref[...], re2, z,
         ego0[:, 32:], im1_ref[...], im2, z], axis=1)


def _dense1(ego0, e1, side1, w1t, b1, w2t, b2, re1, im1):
    row = lambda i: (i, 0)
    full = lambda i: (0, 0)
    return pl.pallas_call(
        _dense1_body,
        grid=(N_BLKS,),
        in_specs=[
            pl.BlockSpec((R_BLK, 64), row),
            pl.BlockSpec((1, R_BLK, 16), lambda i: (0, i, 0)),
            pl.BlockSpec((1, R_BLK, 16), lambda i: (1, i, 0)),
            pl.BlockSpec((1, R_BLK, 16), lambda i: (0, i, 0)),
            pl.BlockSpec((1, R_BLK, 16), lambda i: (1, i, 0)),
            pl.BlockSpec((32, 16), full),
            pl.BlockSpec((1, 16), full),
            pl.BlockSpec((32, 16), full),
            pl.BlockSpec((1, 16), full),
            pl.BlockSpec((R_BLK, 16), row),
            pl.BlockSpec((R_BLK, 16), row),
        ],
        out_specs=pl.BlockSpec((R_BLK, 128), row),
        out_shape=jax.ShapeDtypeStruct((N, 128), jnp.float32),
    )(ego0, e1, e1, side1, side1, w1t, b1, w2t, b2, re1, im1)


def _loss_body(pos_ref, neg_ref, l2u_ref, l2p_ref, l2n_ref, out_ref):
    pos = jnp.sum(pos_ref[...], axis=1)
    neg = jnp.sum(neg_ref[...], axis=1)
    x = pos - neg
    nls = jnp.where(x >= 0, jnp.log1p(jnp.exp(-x)), -x + jnp.log1p(jnp.exp(x)))
    l2 = (jnp.mean(jnp.sum(l2u_ref[...], axis=1))
          + jnp.mean(jnp.sum(l2p_ref[...], axis=1))
          + jnp.mean(jnp.sum(l2n_ref[...], axis=1))) * 0.5
    out_ref[...] = (jnp.mean(nls) + CF_LAMBDA * l2).reshape(1, 1)


def _loss(pos_p, neg_p, l2u, l2p, l2n):
    return pl.pallas_call(
        _loss_body,
        out_shape=jax.ShapeDtypeStruct((1, 1), jnp.float32),
    )(pos_p, neg_p, l2u, l2p, l2n)


_CHUNK = 128
_SUP = 1024
_N_TILES = 16
_E_PAD = 819200
_PER_TILE = _E_PAD // _N_TILES
_N_SUP = _PER_TILE // _SUP
_IDX_ROWS = _E_PAD // _CHUNK
_VAL_ROWS = _E_PAD // 16
_ROWS_PER_TILE = 3128


def _copy_rows(src_ref, dst_ref, s):
    @pl.when(s < _N_TILES - 1)
    def _():
        r0 = pl.multiple_of(s * _ROWS_PER_TILE, 8)
        pltpu.sync_copy(src_ref.at[pl.ds(r0, _ROWS_PER_TILE)],
                        dst_ref.at[pl.ds(r0, _ROWS_PER_TILE)])

    @pl.when(s == _N_TILES - 1)
    def _():
        r0 = (_N_TILES - 1) * _ROWS_PER_TILE
        pltpu.sync_copy(src_ref.at[pl.ds(r0, N - r0)],
                        dst_ref.at[pl.ds(r0, N - r0)])


def _make_spmm_body(w):
    n_pass = w // 32

    def body(tabP, src2d, dst2d, val16, zeros, outP,
             srcA, dstA, valA, srcB, dstB, valB, rowsA, rowsB, accum,
             gsem, ssem, isem):
        c = lax.axis_index("c")
        s = lax.axis_index("s")
        bufA = (srcA, dstA, valA, rowsA)
        bufB = (srcB, dstB, valB, rowsB)
        base_i = s * (_PER_TILE // _CHUNK)
        base_v = s * (_PER_TILE // 16)

        def rslice(rows, j):
            return rows.at[pl.ds(pl.multiple_of(j * _CHUNK, _CHUNK), _CHUNK)]

        def one_pass(tab_q, out_q):
            _copy_rows(zeros, accum, s)
            plsc.subcore_barrier()

            def superchunk(sc, X, Y, first=False, last=False):
                (srcX, dstX, valX, rowsX) = X
                (srcY, dstY, valY, rowsY) = Y
                if not first:

                    def sdrain0(j, carry):
                        pltpu.make_async_copy(
                            rslice(rowsY, j), accum.at[dstY.at[j]],
                            ssem.at[j]).wait()
                        return carry

                    lax.fori_loop(0, _SUP // _CHUNK, sdrain0, 0)
                if not last:
                    ri = pl.multiple_of(base_i + (sc + 1) * (_SUP // _CHUNK), 8)
                    rv = pl.multiple_of(base_v + (sc + 1) * (_SUP // 16), 8)
                    pltpu.async_copy(src2d.at[pl.ds(ri, _SUP // _CHUNK)], srcY, isem)
                    pltpu.async_copy(dst2d.at[pl.ds(ri, _SUP // _CHUNK)], dstY, isem)
                    pltpu.async_copy(val16.at[pl.ds(rv, _SUP // 16)], valY, isem)

                def jbody(j, carry):
                    pltpu.make_async_copy(
                        tab_q.at[srcX.at[j]], rslice(rowsX, j),
                        gsem.at[j]).wait()

                    def grp(g2, carry2):
                        g = j * 8 + g2
                        vv = valX[g]
                        for i in range(16):
                            k = g * 16 + i
                            v = vv[i]
                            rowsX[k, pl.ds(0, 16)] = rowsX[k, pl.ds(0, 16)] * v
                        return carry2

                    lax.fori_loop(0, 8, grp, 0)
                    pltpu.make_async_copy(
                        rslice(rowsX, j), accum.at[dstX.at[j]],
                        ssem.at[j]).start(add=True)
                    if not last:
                        @pl.when(j == 1)
                        def _():
                            pltpu.make_async_copy(
                                src2d.at[pl.ds(ri, _SUP // _CHUNK)], srcY, isem).wait()
                            pltpu.make_async_copy(
                                dst2d.at[pl.ds(ri, _SUP // _CHUNK)], dstY, isem).wait()
                            pltpu.make_async_copy(
                                val16.at[pl.ds(rv, _SUP // 16)], valY, isem).wait()

                        @pl.when(j >= 1)
                        def _():
                            jp = j - 1
                            pltpu.async_copy(
                                tab_q.at[srcY.at[jp]], rslice(rowsY, jp),
                                gsem.at[jp])
                    return carry

                lax.fori_loop(0, _SUP // _CHUNK, jbody, 0)
                if not last:
                    _last = _SUP // _CHUNK - 1
                    pltpu.async_copy(
                        tab_q.at[srcY.at[_last]], rslice(rowsY, _last),
                        gsem.at[_last])

            r0 = pl.multiple_of(base_i, 8)
            pltpu.sync_copy(src2d.at[pl.ds(r0, _SUP // _CHUNK)], srcA)
            pltpu.sync_copy(dst2d.at[pl.ds(r0, _SUP // _CHUNK)], dstA)
            pltpu.sync_copy(
                val16.at[pl.ds(pl.multiple_of(base_v, 8), _SUP // 16)], valA)

            def gfire0(j, carry):
                pltpu.async_copy(
                    tab_q.at[srcA.at[j]], rslice(rowsA, j), gsem.at[j])
                return carry

            lax.fori_loop(0, _SUP // _CHUNK, gfire0, 0)

            superchunk(0, bufA, bufB, first=True)

            def dbl(dsc, carry):
                superchunk(2 * dsc + 1, bufB, bufA)
                superchunk(2 * dsc + 2, bufA, bufB)
                return carry

            lax.fori_loop(0, (_N_SUP - 2) // 2, dbl, 0)
            superchunk(_N_SUP - 1, bufB, bufA, last=True)

            def sdrain(j, carry):
                pltpu.make_async_copy(
                    rslice(rowsB, j), accum.at[dstB.at[j]], ssem.at[j]).wait()
                return carry

            lax.fori_loop(0, _SUP // _CHUNK, sdrain, 0)

            plsc.subcore_barrier()
            _copy_rows(accum, out_q, s)

        for p in range(n_pass):
            q = c * n_pass + p
            one_pass(tabP.at[q], outP.at[q])
            if p + 1 < n_pass:
                plsc.subcore_barrier()

    return body


@functools.partial(jax.jit, static_argnames=("w",))
def _spmm_sc(tabP, src2d, dst2d, val16, w):
    mesh = plsc.VectorSubcoreMesh(core_axis_name="c", subcore_axis_name="s")
    zeros = jnp.zeros((N, 16), jnp.float32)
    f = pl.kernel(
        _make_spmm_body(w),
        out_type=jax.ShapeDtypeStruct((w // 16, N, 16), jnp.float32),
        mesh=mesh,
        scratch_types=[
            pltpu.VMEM((_SUP // _CHUNK, _CHUNK), jnp.int32),
            pltpu.VMEM((_SUP // _CHUNK, _CHUNK), jnp.int32),
            pltpu.VMEM((_SUP // 16, 16), jnp.float32),
            pltpu.VMEM((_SUP // _CHUNK, _CHUNK), jnp.int32),
            pltpu.VMEM((_SUP // _CHUNK, _CHUNK), jnp.int32),
            pltpu.VMEM((_SUP // 16, 16), jnp.float32),
            pltpu.VMEM((_SUP, 16), jnp.float32),
            pltpu.VMEM((_SUP, 16), jnp.float32),
            pltpu.VMEM_SHARED((N, 16), jnp.float32),
            pltpu.SemaphoreType.DMA((_SUP // _CHUNK,)),
            pltpu.SemaphoreType.DMA((_SUP // _CHUNK,)),
            pltpu.SemaphoreType.DMA,
        ],
        compiler_params=pltpu.CompilerParams(use_tc_tiling_on_sc=False),
    )
    return f(tabP, src2d, dst2d, val16, zeros)


def _pad_edges(src, dst, val):
    pad = _E_PAD - E
    z = jnp.zeros((pad,), jnp.int32)
    src2d = jnp.concatenate([src, z]).reshape(_IDX_ROWS, _CHUNK)
    dst2d = jnp.concatenate([dst, z]).reshape(_IDX_ROWS, _CHUNK)
    val16 = jnp.concatenate(
        [val, jnp.zeros((pad,), jnp.float32)]).reshape(_VAL_ROWS, 16)
    return src2d, dst2d, val16


def _gather_partials(feat, uid, pid, nid):
    u = feat[uid]
    p = feat[pid]
    n = feat[nid]
    pos_p = (u * p).reshape(B, 8, 16).sum(axis=1)
    neg_p = (u * n).reshape(B, 8, 16).sum(axis=1)
    l2u = (u[:, :64] ** 2).reshape(B, 4, 16).sum(axis=1)
    l2p = (p[:, :64] ** 2).reshape(B, 4, 16).sum(axis=1)
    l2n = (n[:, :64] ** 2).reshape(B, 4, 16).sum(axis=1)
    return pos_p, neg_p, l2u, l2p, l2n


def kernel(user_ids, item_pos_ids, item_neg_ids, entity_user_embed,
           edge_src, edge_dst, edge_val,
           W1_0, b1_0, W2_0, b2_0, W1_1, b1_1, W2_1, b2_1):
    ego0 = entity_user_embed
    src2d, dst2d, val16 = _pad_edges(edge_src, edge_dst, edge_val)
    tab0 = jnp.stack([ego0[:, 0:16], ego0[:, 16:32], ego0[:, 32:48], ego0[:, 48:64]])
    side0 = _spmm_sc(tab0, src2d, dst2d, val16, 64)
    e1, re1, im1 = _dense0(
        ego0, side0, W1_0.T, b1_0.reshape(1, 32), W2_0.T, b2_0.reshape(1, 32))
    side1 = _spmm_sc(e1, src2d, dst2d, val16, 32)
    feat = _dense1(ego0, e1, side1,
                   W1_1.T, b1_1.reshape(1, 16), W2_1.T, b2_1.reshape(1, 16),
                   re1, im1)
    pos_p, neg_p, l2u, l2p, l2n = _gather_partials(
        feat, user_ids, item_pos_ids, item_neg_ids)
    out = _loss(pos_p, neg_p, l2u, l2p, l2n)
    return out.reshape(())


# speedup vs baseline: 1.0429x; 1.0073x over previous
import functools

import jax
import jax.numpy as jnp
from jax import lax
from jax.experimental import pallas as pl
from jax.experimental.pallas import tpu as pltpu
from jax.experimental.pallas import tpu_sc as plsc

N_USERS = 10000
N_ENTITIES = 40000
N = N_USERS + N_ENTITIES
E = 800000
EMBED = 64
CF_LAMBDA = 1e-05
B = 1024

R_BLK = 5000
N_BLKS = N // R_BLK


def _leaky(x):
    return jnp.where(x >= 0, x, 0.01 * x)


def _l2n(x):
    n = jnp.sqrt(jnp.sum(x * x, axis=1, keepdims=True))
    return x / jnp.maximum(n, 1e-12)


def _dense0_body(ego_ref, p0_ref, p1_ref, p2_ref, p3_ref,
                 w1_ref, b1_ref, w2_ref, b2_ref,
                 e1_ref, re1_ref, im1_ref):
    ego = ego_ref[...]
    side = jnp.concatenate([p0_ref[0], p1_ref[0], p2_ref[0], p3_ref[0]], axis=1)
    h1 = _leaky(jnp.dot(ego + side, w1_ref[...],
                        preferred_element_type=jnp.float32) + b1_ref[...])
    h2 = _leaky(jnp.dot(ego * side, w2_ref[...],
                        preferred_element_type=jnp.float32) + b2_ref[...])
    e1 = h1 + h2
    eL = e1[:, :16]
    eR = e1[:, 16:]
    e1_ref[...] = jnp.stack([eL, eR], axis=0)
    re1_ref[...] = _l2n(eL)
    im1_ref[...] = _l2n(eR)


def _dense0(ego, side0, w1t, b1, w2t, b2):
    row = lambda i: (i, 0)
    full = lambda i: (0, 0)
    return pl.pallas_call(
        _dense0_body,
        grid=(N_BLKS,),
        in_specs=[
            pl.BlockSpec((R_BLK, 64), row),
            pl.BlockSpec((1, R_BLK, 16), lambda i: (0, i, 0)),
            pl.BlockSpec((1, R_BLK, 16), lambda i: (1, i, 0)),
            pl.BlockSpec((1, R_BLK, 16), lambda i: (2, i, 0)),
            pl.BlockSpec((1, R_BLK, 16), lambda i: (3, i, 0)),
            pl.BlockSpec((64, 32), full),
            pl.BlockSpec((1, 32), full),
            pl.BlockSpec((64, 32), full),
            pl.BlockSpec((1, 32), full),
        ],
        out_specs=[
            pl.BlockSpec((2, R_BLK, 16), lambda i: (0, i, 0)),
            pl.BlockSpec((R_BLK, 16), row),
            pl.BlockSpec((R_BLK, 16), row),
        ],
        out_shape=[
            jax.ShapeDtypeStruct((2, N, 16), jnp.float32),
            jax.ShapeDtypeStruct((N, 16), jnp.float32),
            jax.ShapeDtypeStruct((N, 16), jnp.float32),
        ],
    )(ego, side0, side0, side0, side0, w1t, b1, w2t, b2)


def _dense1_body(ego0_ref, ea_ref, eb_ref, sa_ref, sb_ref,
                 w1_ref, b1_ref, w2_ref, b2_ref, re1_ref, im1_ref, feat_ref):
    e1 = jnp.concatenate([ea_ref[0], eb_ref[0]], axis=1)
    s1 = jnp.concatenate([sa_ref[0], sb_ref[0]], axis=1)
    h1 = _leaky(jnp.dot(e1 + s1, w1_ref[...],
                        preferred_element_type=jnp.float32) + b1_ref[...])
    h2 = _leaky(jnp.dot(e1 * s1, w2_ref[...],
                        preferred_element_type=jnp.float32) + b2_ref[...])
    e2 = h1 + h2
    re2 = _l2n(e2[:, :8])
    im2 = _l2n(e2[:, 8:])
    ego0 = ego0_ref[...]
    z = jnp.zeros((ego0.shape[0], 8), jnp.float32)
    feat_ref[...] = jnp.concatenate(
        [ego0[:, :32], re1_ref[...], re2, z,
         ego0[:, 32:], im1_ref[...], im2, z], axis=1)


def _dense1(ego0, e1, side1, w1t, b1, w2t, b2, re1, im1):
    row = lambda i: (i, 0)
    full = lambda i: (0, 0)
    return pl.pallas_call(
        _dense1_body,
        grid=(N_BLKS,),
        in_specs=[
            pl.BlockSpec((R_BLK, 64), row),
            pl.BlockSpec((1, R_BLK, 16), lambda i: (0, i, 0)),
            pl.BlockSpec((1, R_BLK, 16), lambda i: (1, i, 0)),
            pl.BlockSpec((1, R_BLK, 16), lambda i: (0, i, 0)),
            pl.BlockSpec((1, R_BLK, 16), lambda i: (1, i, 0)),
            pl.BlockSpec((32, 16), full),
            pl.BlockSpec((1, 16), full),
            pl.BlockSpec((32, 16), full),
            pl.BlockSpec((1, 16), full),
            pl.BlockSpec((R_BLK, 16), row),
            pl.BlockSpec((R_BLK, 16), row),
        ],
        out_specs=pl.BlockSpec((R_BLK, 128), row),
        out_shape=jax.ShapeDtypeStruct((N, 128), jnp.float32),
    )(ego0, e1, e1, side1, side1, w1t, b1, w2t, b2, re1, im1)


def _loss_body(pos_ref, neg_ref, l2u_ref, l2p_ref, l2n_ref, out_ref):
    pos = jnp.sum(pos_ref[...], axis=1)
    neg = jnp.sum(neg_ref[...], axis=1)
    x = pos - neg
    nls = jnp.where(x >= 0, jnp.log1p(jnp.exp(-x)), -x + jnp.log1p(jnp.exp(x)))
    l2 = (jnp.mean(jnp.sum(l2u_ref[...], axis=1))
          + jnp.mean(jnp.sum(l2p_ref[...], axis=1))
          + jnp.mean(jnp.sum(l2n_ref[...], axis=1))) * 0.5
    out_ref[...] = (jnp.mean(nls) + CF_LAMBDA * l2).reshape(1, 1)


def _loss(pos_p, neg_p, l2u, l2p, l2n):
    return pl.pallas_call(
        _loss_body,
        out_shape=jax.ShapeDtypeStruct((1, 1), jnp.float32),
    )(pos_p, neg_p, l2u, l2p, l2n)


_CHUNK = 128
_SUP = 1024
_N_TILES = 16
_E_PAD = 819200
_PER_TILE = _E_PAD // _N_TILES
_N_SUP = _PER_TILE // _SUP
_IDX_ROWS = _E_PAD // _CHUNK
_VAL_ROWS = _E_PAD // 16
_ROWS_PER_TILE = 3128


def _copy_rows(src_ref, dst_ref, s):
    @pl.when(s < _N_TILES - 1)
    def _():
        r0 = pl.multiple_of(s * _ROWS_PER_TILE, 8)
        pltpu.sync_copy(src_ref.at[pl.ds(r0, _ROWS_PER_TILE)],
                        dst_ref.at[pl.ds(r0, _ROWS_PER_TILE)])

    @pl.when(s == _N_TILES - 1)
    def _():
        r0 = (_N_TILES - 1) * _ROWS_PER_TILE
        pltpu.sync_copy(src_ref.at[pl.ds(r0, N - r0)],
                        dst_ref.at[pl.ds(r0, N - r0)])


def _make_spmm_body(w):
    n_pass = w // 32

    def body(tabP, src2d, dst2d, val16, zeros, outP,
             srcA, dstA, valA, srcB, dstB, valB, rowsA, rowsB, accum,
             gsem, ssem, isem):
        c = lax.axis_index("c")
        s = lax.axis_index("s")
        bufA = (srcA, dstA, valA, rowsA)
        bufB = (srcB, dstB, valB, rowsB)
        base_i = s * (_PER_TILE // _CHUNK)
        base_v = s * (_PER_TILE // 16)

        def rslice(rows, j):
            return rows.at[pl.ds(pl.multiple_of(j * _CHUNK, _CHUNK), _CHUNK)]

        def one_pass(tab_q, out_q):
            _copy_rows(zeros, accum, s)
            plsc.subcore_barrier()

            def superchunk(sc, X, Y, first=False, last=False):
                (srcX, dstX, valX, rowsX) = X
                (srcY, dstY, valY, rowsY) = Y
                if not first:

                    def sdrain0(j, carry):
                        pltpu.make_async_copy(
                            rslice(rowsY, j), accum.at[dstY.at[j]],
                            ssem.at[j]).wait()
                        return carry

                    lax.fori_loop(0, _SUP // _CHUNK, sdrain0, 0)
                if not last:
                    ri = pl.multiple_of(base_i + (sc + 1) * (_SUP // _CHUNK), 8)
                    rv = pl.multiple_of(base_v + (sc + 1) * (_SUP // 16), 8)
                    pltpu.async_copy(src2d.at[pl.ds(ri, _SUP // _CHUNK)], srcY, isem)
                    pltpu.async_copy(dst2d.at[pl.ds(ri, _SUP // _CHUNK)], dstY, isem)
                    pltpu.async_copy(val16.at[pl.ds(rv, _SUP // 16)], valY, isem)

                def jbody(j, carry):
                    pltpu.make_async_copy(
                        tab_q.at[srcX.at[j]], rslice(rowsX, j),
                        gsem.at[j]).wait()

                    def grp(g2, carry2):
                        g = j * 8 + g2
                        vv = valX[g]
                        for i in range(16):
                            k = g * 16 + i
                            v = vv[i]
                            rowsX[k, pl.ds(0, 16)] = rowsX[k, pl.ds(0, 16)] * v
                        return carry2

                    lax.fori_loop(0, 8, grp, 0)
                    pltpu.make_async_copy(
                        rslice(rowsX, j), accum.at[dstX.at[j]],
                        ssem.at[j]).start(add=True)
                    if not last:
                        @pl.when(j == 1)
                        def _():
                            pltpu.make_async_copy(
                                src2d.at[pl.ds(ri, _SUP // _CHUNK)], srcY, isem).wait()
                            pltpu.make_async_copy(
                                dst2d.at[pl.ds(ri, _SUP // _CHUNK)], dstY, isem).wait()
                            pltpu.make_async_copy(
                                val16.at[pl.ds(rv, _SUP // 16)], valY, isem).wait()

                        @pl.when(j >= 1)
                        def _():
                            jp = j - 1
                            pltpu.async_copy(
                                tab_q.at[srcY.at[jp]], rslice(rowsY, jp),
                                gsem.at[jp])
                    return carry

                lax.fori_loop(0, _SUP // _CHUNK, jbody, 0)
                if not last:
                    _last = _SUP // _CHUNK - 1
                    pltpu.async_copy(
                        tab_q.at[srcY.at[_last]], rslice(rowsY, _last),
                        gsem.at[_last])

            r0 = pl.multiple_of(base_i, 8)
            pltpu.sync_copy(src2d.at[pl.ds(r0, _SUP // _CHUNK)], srcA)
            pltpu.sync_copy(dst2d.at[pl.ds(r0, _SUP // _CHUNK)], dstA)
            pltpu.sync_copy(
                val16.at[pl.ds(pl.multiple_of(base_v, 8), _SUP // 16)], valA)

            def gfire0(j, carry):
                pltpu.async_copy(
                    tab_q.at[srcA.at[j]], rslice(rowsA, j), gsem.at[j])
                return carry

            lax.fori_loop(0, _SUP // _CHUNK, gfire0, 0)

            superchunk(0, bufA, bufB, first=True)

            def dbl(dsc, carry):
                superchunk(2 * dsc + 1, bufB, bufA)
                superchunk(2 * dsc + 2, bufA, bufB)
                return carry

            lax.fori_loop(0, (_N_SUP - 2) // 2, dbl, 0)
            superchunk(_N_SUP - 1, bufB, bufA, last=True)

            def sdrain(j, carry):
                pltpu.make_async_copy(
                    rslice(rowsB, j), accum.at[dstB.at[j]], ssem.at[j]).wait()
                return carry

            lax.fori_loop(0, _SUP // _CHUNK, sdrain, 0)

            plsc.subcore_barrier()
            _copy_rows(accum, out_q, s)

        for p in range(n_pass):
            q = c * n_pass + p
            one_pass(tabP.at[q], outP.at[q])
            if p + 1 < n_pass:
                plsc.subcore_barrier()

    return body


@functools.partial(jax.jit, static_argnames=("w",))
def _spmm_sc(tabP, src2d, dst2d, val16, w):
    mesh = plsc.VectorSubcoreMesh(core_axis_name="c", subcore_axis_name="s")
    zeros = jnp.zeros((N, 16), jnp.float32)
    f = pl.kernel(
        _make_spmm_body(w),
        out_type=jax.ShapeDtypeStruct((w // 16, N, 16), jnp.float32),
        mesh=mesh,
        scratch_types=[
            pltpu.VMEM((_SUP // _CHUNK, _CHUNK), jnp.int32),
            pltpu.VMEM((_SUP // _CHUNK, _CHUNK), jnp.int32),
            pltpu.VMEM((_SUP // 16, 16), jnp.float32),
            pltpu.VMEM((_SUP // _CHUNK, _CHUNK), jnp.int32),
            pltpu.VMEM((_SUP // _CHUNK, _CHUNK), jnp.int32),
            pltpu.VMEM((_SUP // 16, 16), jnp.float32),
            pltpu.VMEM((_SUP, 16), jnp.float32),
            pltpu.VMEM((_SUP, 16), jnp.float32),
            pltpu.VMEM_SHARED((N, 16), jnp.float32),
            pltpu.SemaphoreType.DMA((_SUP // _CHUNK,)),
            pltpu.SemaphoreType.DMA((_SUP // _CHUNK,)),
            pltpu.SemaphoreType.DMA,
        ],
        compiler_params=pltpu.CompilerParams(use_tc_tiling_on_sc=False),
    )
    return f(tabP, src2d, dst2d, val16, zeros)


def _pad_edges(src, dst, val):
    pad = _E_PAD - E
    z = jnp.zeros((pad,), jnp.int32)
    src2d = jnp.concatenate([src, z]).reshape(_IDX_ROWS, _CHUNK)
    dst2d = jnp.concatenate([dst, z]).reshape(_IDX_ROWS, _CHUNK)
    val16 = jnp.concatenate(
        [val, jnp.zeros((pad,), jnp.float32)]).reshape(_VAL_ROWS, 16)
    return src2d, dst2d, val16


def _gather_partials(feat, uid, pid, nid):
    u = feat[uid]
    p = feat[pid]
    n = feat[nid]
    pos_p = (u * p).reshape(B, 8, 16).sum(axis=1)
    neg_p = (u * n).reshape(B, 8, 16).sum(axis=1)
    l2u = (u[:, :64] ** 2).reshape(B, 4, 16).sum(axis=1)
    l2p = (p[:, :64] ** 2).reshape(B, 4, 16).sum(axis=1)
    l2n = (n[:, :64] ** 2).reshape(B, 4, 16).sum(axis=1)
    return pos_p, neg_p, l2u, l2p, l2n


def kernel(user_ids, item_pos_ids, item_neg_ids, entity_user_embed,
           edge_src, edge_dst, edge_val,
           W1_0, b1_0, W2_0, b2_0, W1_1, b1_1, W2_1, b2_1):
    ego0 = entity_user_embed
    src2d, dst2d, val16 = _pad_edges(edge_src, edge_dst, edge_val)
    tab0 = jnp.stack([ego0[:, 0:16], ego0[:, 16:32], ego0[:, 32:48], ego0[:, 48:64]])
    side0 = _spmm_sc(tab0, src2d, dst2d, val16, 64)
    e1, re1, im1 = _dense0(
        ego0, side0, W1_0.T, b1_0.reshape(1, 32), W2_0.T, b2_0.reshape(1, 32))
    side1 = _spmm_sc(e1, src2d, dst2d, val16, 32)
    feat = _dense1(ego0, e1, side1,
                   W1_1.T, b1_1.reshape(1, 16), W2_1.T, b2_1.reshape(1, 16),
                   re1, im1)
    pos_p, neg_p, l2u, l2p, l2n = _gather_partials(
        feat, user_ids, item_pos_ids, item_neg_ids)
    out = _loss(pos_p, neg_p, l2u, l2p, l2n)
    return out.reshape(())
